# baseline (speedup 1.0000x reference)
import functools

import jax
import jax.numpy as jnp
from jax import lax
from jax.experimental import pallas as pl
from jax.experimental.pallas import tpu as pltpu
from jax.experimental.pallas import tpu_sc as plsc

N = 10000
F = 128
H1 = 32
H2 = 16
E = 320000

NC, NS, LANES = 2, 16, 16
CHUNK = 128
CPW0 = 120
CPW1 = 40
CROWS = NS * (CPW0 + CPW1)
CROWS_AL = CROWS + (CPW0 - CPW1)
EPAD = CHUNK * CROWS_AL
NPAD = 10112
RPT = NPAD // NS
KBUF = 8


def _seg_sum_sc(feat, src2d, dst2d, nf):
    mesh = plsc.VectorSubcoreMesh(core_axis_name="c", subcore_axis_name="s")

    @functools.partial(
        pl.kernel,
        out_type=jax.ShapeDtypeStruct((NC, NPAD, nf), jnp.float32),
        mesh=mesh,
        compiler_params=pltpu.CompilerParams(use_tc_tiling_on_sc=False),
        scratch_types=(
            [pltpu.VMEM((CPW0, CHUNK), jnp.int32),
             pltpu.VMEM((CPW0, CHUNK), jnp.int32)]
            + [pltpu.VMEM((CHUNK, nf), jnp.float32)
               for _ in range(KBUF)]
            + [pltpu.VMEM((RPT, nf), jnp.float32),
               pltpu.VMEM_SHARED((NPAD, nf), jnp.float32),
               pltpu.SemaphoreType.DMA,
               pltpu.SemaphoreType.DMA]
        ),
    )
    def body(feat_hbm, src_hbm, dst_hbm, out_hbm, *scr):
        src_v, dst_v = scr[0], scr[1]
        bufs = scr[2:2 + KBUF]
        zb_v, acc_sh, gsem, ssem = scr[2 + KBUF:]
        c = lax.axis_index("c")
        s = lax.axis_index("s")
        cbase = jnp.where(c == 0, s * CPW0, NS * CPW0 + s * CPW1)
        ngroups = jnp.where(c == 0, CPW0 // KBUF, CPW1 // KBUF)

        def zrow(i, carry):
            for j in range(nf // LANES):
                zb_v[i, pl.ds(j * LANES, LANES)] = jnp.zeros((LANES,),
                                                             jnp.float32)
            return carry
        lax.fori_loop(0, RPT, zrow, 0)
        pltpu.sync_copy(zb_v, acc_sh.at[pl.ds(s * RPT, RPT)])
        plsc.subcore_barrier()

        pltpu.sync_copy(src_hbm.at[pl.ds(cbase, CPW0)], src_v)
        pltpu.sync_copy(dst_hbm.at[pl.ds(cbase, CPW0)], dst_v)

        def group(g, carry):
            base = g * KBUF
            gd = [pltpu.async_copy(feat_hbm.at[src_v.at[base + j]],
                                   bufs[j], gsem)
                  for j in range(KBUF)]
            for d in gd:
                d.wait()
            sd = [pltpu.async_copy(bufs[j], acc_sh.at[dst_v.at[base + j]],
                                   ssem, add=True)
                  for j in range(KBUF)]
            for d in sd:
                d.wait()
            return carry
        lax.fori_loop(0, ngroups, group, 0)

        plsc.subcore_barrier()
        pltpu.sync_copy(acc_sh.at[pl.ds(s * RPT, RPT)], zb_v)
        pltpu.sync_copy(zb_v, out_hbm.at[c, pl.ds(s * RPT, RPT)])

    return body(feat, src2d, dst2d)


def _mm1(x, W1):
    def body(x_ref, w_ref, o_ref):
        o_ref[...] = jnp.dot(x_ref[...], w_ref[...],
                             preferred_element_type=jnp.float32)
    return pl.pallas_call(
        body,
        grid=(10,),
        in_specs=[pl.BlockSpec((1000, F), lambda i: (i, 0)),
                  pl.BlockSpec((F, H1), lambda i: (0, 0))],
        out_specs=pl.BlockSpec((1000, H1), lambda i: (i, 0)),
        out_shape=jax.ShapeDtypeStruct((N, H1), jnp.float32),
    )(x, W1)


def _mm2(p, W2):
    def body(p_ref, w_ref, o_ref):
        h = jnp.maximum(p_ref[0] + p_ref[1], 0.0)
        o_ref[...] = jnp.dot(h, w_ref[...],
                             preferred_element_type=jnp.float32)
    return pl.pallas_call(
        body,
        grid=(10,),
        in_specs=[pl.BlockSpec((2, 1000, H1), lambda i: (0, i, 0)),
                  pl.BlockSpec((H1, H2), lambda i: (0, 0))],
        out_specs=pl.BlockSpec((1000, H2), lambda i: (i, 0)),
        out_shape=jax.ShapeDtypeStruct((N, H2), jnp.float32),
    )(p, W2)


def _combine(p):
    def body(p_ref, o_ref):
        o_ref[...] = p_ref[0] + p_ref[1]
    return pl.pallas_call(
        body,
        grid=(8,),
        in_specs=[pl.BlockSpec((2, 1264, H2), lambda i: (0, i, 0))],
        out_specs=pl.BlockSpec((1264, H2), lambda i: (i, 0)),
        out_shape=jax.ShapeDtypeStruct((NPAD, H2), jnp.float32),
    )(p)


def _decoder(z):
    BR, BC = 1024, 2048
    GR = (N + BR - 1) // BR
    GC = (N + BC - 1) // BC

    def body(zr_ref, zc_ref, o_ref):
        acc = lax.dot_general(zr_ref[...], zc_ref[...],
                              (((1,), (1,)), ((), ())),
                              preferred_element_type=jnp.float32)
        o_ref[...] = jnp.tanh(acc)

    return pl.pallas_call(
        body,
        grid=(GR, GC),
        in_specs=[pl.BlockSpec((BR, H2), lambda i, j: (i, 0)),
                  pl.BlockSpec((BC, H2), lambda i, j: (j, 0))],
        out_specs=pl.BlockSpec((BR, BC), lambda i, j: (i, j)),
        out_shape=jax.ShapeDtypeStruct((N, N), jnp.float32),
    )(z, z)


def kernel(x, edge_index, W1, W2):
    src = edge_index[0]
    dst = edge_index[1]
    pad = EPAD - E
    src_p = jnp.concatenate(
        [src, jnp.zeros((pad,), jnp.int32)]).reshape(CROWS_AL, CHUNK)
    dst_p = jnp.concatenate(
        [dst, jnp.full((pad,), N, jnp.int32)]).reshape(CROWS_AL, CHUNK)

    h0 = _mm1(x, W1)
    p1 = _seg_sum_sc(h0, src_p, dst_p, H1)
    z0 = _mm2(p1, W2)
    p2 = _seg_sum_sc(z0, src_p, dst_p, H2)
    z = _combine(p2)
    return _decoder(z)


# speedup vs baseline: 1.0646x; 1.0646x over previous
import functools

import jax
import jax.numpy as jnp
from jax import lax
from jax.experimental import pallas as pl
from jax.experimental.pallas import tpu as pltpu
from jax.experimental.pallas import tpu_sc as plsc

N = 10000
F = 128
H1 = 32
H2 = 16
E = 320000

NC, NS, LANES = 2, 16, 16
CHUNK = 128
CPW0 = 136
CPW1 = 24
CROWS = NS * (CPW0 + CPW1)
CROWS_AL = CROWS + (CPW0 - CPW1)
EPAD = CHUNK * CROWS_AL
NPAD = 10112
RPT = NPAD // NS
KBUF = 8


def _seg_sum_sc(feat, src2d, dst2d, nf):
    mesh = plsc.VectorSubcoreMesh(core_axis_name="c", subcore_axis_name="s")

    @functools.partial(
        pl.kernel,
        out_type=jax.ShapeDtypeStruct((NC, NPAD, nf), jnp.float32),
        mesh=mesh,
        compiler_params=pltpu.CompilerParams(use_tc_tiling_on_sc=False),
        scratch_types=(
            [pltpu.VMEM((CPW0, CHUNK), jnp.int32),
             pltpu.VMEM((CPW0, CHUNK), jnp.int32)]
            + [pltpu.VMEM((CHUNK, nf), jnp.float32)
               for _ in range(KBUF)]
            + [pltpu.VMEM((RPT, nf), jnp.float32),
               pltpu.VMEM_SHARED((NPAD, nf), jnp.float32),
               pltpu.SemaphoreType.DMA,
               pltpu.SemaphoreType.DMA]
        ),
    )
    def body(feat_hbm, src_hbm, dst_hbm, out_hbm, *scr):
        src_v, dst_v = scr[0], scr[1]
        bufs = scr[2:2 + KBUF]
        zb_v, acc_sh, gsem, ssem = scr[2 + KBUF:]
        c = lax.axis_index("c")
        s = lax.axis_index("s")
        cbase = jnp.where(c == 0, s * CPW0, NS * CPW0 + s * CPW1)
        ngroups = jnp.where(c == 0, CPW0 // KBUF, CPW1 // KBUF)

        def zrow(i, carry):
            for j in range(nf // LANES):
                zb_v[i, pl.ds(j * LANES, LANES)] = jnp.zeros((LANES,),
                                                             jnp.float32)
            return carry
        lax.fori_loop(0, RPT, zrow, 0)
        pltpu.sync_copy(zb_v, acc_sh.at[pl.ds(s * RPT, RPT)])
        plsc.subcore_barrier()

        pltpu.sync_copy(src_hbm.at[pl.ds(cbase, CPW0)], src_v)
        pltpu.sync_copy(dst_hbm.at[pl.ds(cbase, CPW0)], dst_v)

        def group(g, carry):
            base = g * KBUF
            gd = [pltpu.async_copy(feat_hbm.at[src_v.at[base + j]],
                                   bufs[j], gsem)
                  for j in range(KBUF)]
            for d in gd:
                d.wait()
            sd = [pltpu.async_copy(bufs[j], acc_sh.at[dst_v.at[base + j]],
                                   ssem, add=True)
                  for j in range(KBUF)]
            for d in sd:
                d.wait()
            return carry
        lax.fori_loop(0, ngroups, group, 0)

        plsc.subcore_barrier()
        pltpu.sync_copy(acc_sh.at[pl.ds(s * RPT, RPT)],
                        out_hbm.at[c, pl.ds(s * RPT, RPT)])

    return body(feat, src2d, dst2d)


def _mm1(x, W1):
    def body(x_ref, w_ref, o_ref):
        o_ref[...] = jnp.dot(x_ref[...], w_ref[...],
                             preferred_element_type=jnp.float32)
    return pl.pallas_call(
        body,
        grid=(10,),
        in_specs=[pl.BlockSpec((1000, F), lambda i: (i, 0)),
                  pl.BlockSpec((F, H1), lambda i: (0, 0))],
        out_specs=pl.BlockSpec((1000, H1), lambda i: (i, 0)),
        out_shape=jax.ShapeDtypeStruct((N, H1), jnp.float32),
    )(x, W1)


def _mm2(p, W2):
    def body(p_ref, w_ref, o_ref):
        h = jnp.maximum(p_ref[0] + p_ref[1], 0.0)
        o_ref[...] = jnp.dot(h, w_ref[...],
                             preferred_element_type=jnp.float32)
    return pl.pallas_call(
        body,
        grid=(10,),
        in_specs=[pl.BlockSpec((2, 1000, H1), lambda i: (0, i, 0)),
                  pl.BlockSpec((H1, H2), lambda i: (0, 0))],
        out_specs=pl.BlockSpec((1000, H2), lambda i: (i, 0)),
        out_shape=jax.ShapeDtypeStruct((N, H2), jnp.float32),
    )(p, W2)


def _combine(p):
    def body(p_ref, o_ref):
        o_ref[...] = p_ref[0] + p_ref[1]
    return pl.pallas_call(
        body,
        grid=(8,),
        in_specs=[pl.BlockSpec((2, 1264, H2), lambda i: (0, i, 0))],
        out_specs=pl.BlockSpec((1264, H2), lambda i: (i, 0)),
        out_shape=jax.ShapeDtypeStruct((NPAD, H2), jnp.float32),
    )(p)


def _decoder(z):
    BR, BC = 2048, 2048
    GR = (N + BR - 1) // BR
    GC = (N + BC - 1) // BC

    def body(zr_ref, zc_ref, o_ref):
        acc = lax.dot_general(zr_ref[...], zc_ref[...],
                              (((1,), (1,)), ((), ())),
                              preferred_element_type=jnp.float32)
        o_ref[...] = jnp.tanh(acc)

    return pl.pallas_call(
        body,
        grid=(GR, GC),
        in_specs=[pl.BlockSpec((BR, H2), lambda i, j: (i, 0)),
                  pl.BlockSpec((BC, H2), lambda i, j: (j, 0))],
        out_specs=pl.BlockSpec((BR, BC), lambda i, j: (i, j)),
        out_shape=jax.ShapeDtypeStruct((N, N), jnp.float32),
    )(z, z)


def kernel(x, edge_index, W1, W2):
    src = edge_index[0]
    dst = edge_index[1]
    pad = EPAD - E
    src_p = jnp.concatenate(
        [src, jnp.zeros((pad,), jnp.int32)]).reshape(CROWS_AL, CHUNK)
    dst_p = jnp.concatenate(
        [dst, jnp.full((pad,), N, jnp.int32)]).reshape(CROWS_AL, CHUNK)

    h0 = _mm1(x, W1)
    p1 = _seg_sum_sc(h0, src_p, dst_p, H1)
    z0 = _mm2(p1, W2)
    p2 = _seg_sum_sc(z0, src_p, dst_p, H2)
    z = _combine(p2)
    return _decoder(z)
